# baseline (speedup 1.0000x reference)
import functools

import jax
import jax.numpy as jnp
from jax import lax
from jax.experimental import pallas as pl
from jax.experimental.pallas import tpu as pltpu
from jax.experimental.pallas import tpu_sc as plsc

_C = 16
_TEX = 1024
_NPIX = 4 * 512 * 512
_NW = 32
_IPW = 512 * 512
_SP = 1024
_SPW = (_NPIX // _SP) // _NW
_CLIP_LO = -123.68
_CLIP_HI = 151.061

_SC_PARAMS = pltpu.CompilerParams(
    needs_layout_passes=False, use_tc_tiling_on_sc=False)

_PROBE_GATHER = True
_PROBE_BLEND = True


def _clip_transpose(tex_tiles):
    mesh = plsc.VectorSubcoreMesh(core_axis_name="c", subcore_axis_name="s")

    @functools.partial(
        pl.kernel,
        mesh=mesh,
        out_type=jax.ShapeDtypeStruct((_TEX, _TEX * _C), jnp.float32),
        compiler_params=_SC_PARAMS,
        scratch_types=[
            pltpu.VMEM((_C, 1024), jnp.float32),
            pltpu.VMEM((_C, 1024), jnp.float32),
            pltpu.VMEM((1024 * _C,), jnp.float32),
            pltpu.VMEM((1024 * _C,), jnp.float32),
            pltpu.SemaphoreType.DMA,
            pltpu.SemaphoreType.DMA,
            pltpu.SemaphoreType.DMA,
            pltpu.SemaphoreType.DMA,
        ],
    )
    def k(tex_hbm, tab_hbm, tba, tbb, oba, obb, isa, isb, osa, osb):
        wid = lax.axis_index("s") * 2 + lax.axis_index("c")
        i16 = lax.iota(jnp.int32, 16)

        def tin(blk, tb, sem, fire):
            yt = blk // 8
            xt = blk % 8
            for c in range(_C):
                d = (pltpu.async_copy if fire else pltpu.make_async_copy)(
                    tex_hbm.at[c, yt, xt], tb.at[c], sem)
                if not fire:
                    d.wait()

        def tout(blk, ob, sem, fire):
            yt = blk // 8
            xt = blk % 8
            for yin in range(8):
                d = (pltpu.async_copy if fire else pltpu.make_async_copy)(
                    ob.at[pl.ds(yin * 2048, 2048)],
                    tab_hbm.at[yt * 8 + yin, pl.ds(xt * 2048, 2048)], sem)
                if not fire:
                    d.wait()

        def transpose_block(tb, ob):
            def tr(g, c2):
                t16 = (g * 16 + i16) * _C
                for c in range(_C):
                    v = jnp.clip(tb[c, pl.ds(g * 16, 16)], _CLIP_LO, _CLIP_HI)
                    plsc.store_scatter(ob, [t16 + c], v)
                return c2

            lax.fori_loop(0, 64, tr, 0)

        def half_iter(blk, tb, ob, isem, osem, it):
            @pl.when(it > 0)
            def _():
                tout(blk, ob, osem, False)

            transpose_block(tb, ob)
            tout(blk, ob, osem, True)
            @pl.when(it < 15)
            def _():
                tin(blk + 2, tb, isem, True)

        def pair(it, carry):
            blk_a = wid * 32 + 2 * it
            tin(blk_a, tba, isa, False)
            half_iter(blk_a, tba, oba, isa, osa, it)
            tin(blk_a + 1, tbb, isb, False)
            half_iter(blk_a + 1, tbb, obb, isb, osb, it)
            return carry

        tin(wid * 32, tba, isa, True)
        tin(wid * 32 + 1, tbb, isb, True)
        lax.fori_loop(0, 16, pair, 0)
        tout(0, oba, osa, False)
        tout(0, obb, osb, False)

    return k(tex_tiles)


def _sc_sample(xf, table):
    mesh = plsc.VectorSubcoreMesh(core_axis_name="c", subcore_axis_name="s")

    @functools.partial(
        pl.kernel,
        mesh=mesh,
        out_type=jax.ShapeDtypeStruct((4, _C, 64, 4, _SP), jnp.float32),
        compiler_params=_SC_PARAMS,
        scratch_types=[
            pltpu.VMEM((8, 2, 128), jnp.float32),
            pltpu.VMEM((8, 2, 128), jnp.float32),
            pltpu.VMEM((8, 128), jnp.int32),
            pltpu.VMEM((8, 128), jnp.int32),
            pltpu.VMEM((8, 128), jnp.int32),
            pltpu.VMEM((8, 128), jnp.int32),
            pltpu.VMEM((_SP,), jnp.float32),
            pltpu.VMEM((_SP,), jnp.float32),
        ] + [pltpu.VMEM((_SP // 2, _C), jnp.float32)] * 8 + [
            pltpu.VMEM((_C * _SP,), jnp.float32),
            pltpu.VMEM((_C * _SP,), jnp.float32),
            pltpu.SemaphoreType.DMA,
            pltpu.SemaphoreType.DMA,
            pltpu.SemaphoreType.DMA,
            pltpu.SemaphoreType.DMA,
            pltpu.SemaphoreType.DMA,
            pltpu.SemaphoreType.DMA,
        ],
    )
    def k(x_hbm, tab_hbm, out_hbm, xba, xbb, i00, i01, i10, i11, wxb, wyb,
          r00a, r00b, r01a, r01b, r10a, r10b, r11a, r11b,
          oba, obb, xsa, xsb, gs0, gs1, osa, osb):
        wid = lax.axis_index("s") * 2 + lax.axis_index("c")
        i16 = lax.iota(jnp.int32, 16)
        ch_base = i16 * _SP
        idx = (i00, i01, i10, i11)
        rows = ((r00a, r00b), (r01a, r01b), (r10a, r10b), (r11a, r11b))

        def coords(s):
            b = s // 256
            rem = s % 256
            return b, rem // 4, rem % 4

        def xmove(s, xb, sem, fire):
            b, yt, xt = coords(s)
            for yin in range(8):
                d = (pltpu.async_copy if fire else pltpu.make_async_copy)(
                    x_hbm.at[b, yt * 8 + yin, xt], xb.at[yin], sem)
                if not fire:
                    d.wait()

        def idx_compute(xb):
            def idx_body(g, c2):
                yin = g // 8
                ks = pl.ds((g % 8) * 16, 16)
                gx = xb[yin, 0, ks]
                gy = xb[yin, 1, ks]
                ix = jnp.clip((gx + 1.0) * 0.5 * (_TEX - 1), 0.0, _TEX - 1.0)
                iy = jnp.clip((gy + 1.0) * 0.5 * (_TEX - 1), 0.0, _TEX - 1.0)
                x0 = ix.astype(jnp.int32)
                y0 = iy.astype(jnp.int32)
                wx = ix - x0.astype(jnp.float32)
                wy = iy - y0.astype(jnp.float32)
                x1 = jnp.minimum(x0 + 1, _TEX - 1)
                y1 = jnp.minimum(y0 + 1, _TEX - 1)
                row0 = y0 * _TEX
                row1 = y1 * _TEX
                j = g // 8
                sl = pl.ds((g % 8) * 16, 16)
                i00[j, sl] = row0 + x0
                i01[j, sl] = row0 + x1
                i10[j, sl] = row1 + x0
                i11[j, sl] = row1 + x1
                fl = pl.ds(g * 16, 16)
                wxb[fl] = wx
                wyb[fl] = wy
                return c2

            lax.fori_loop(0, 64, idx_body, 0)

        def gathers(half, sem, fire):
            for jj in range(4):
                j = half * 4 + jj
                rs = pl.ds(jj * 128, 128)
                for nb in range(4):
                    d = (pltpu.async_copy if fire else pltpu.make_async_copy)(
                        tab_hbm.at[idx[nb].at[j]], rows[nb][half].at[rs], sem)
                    if not fire:
                        d.wait()

        def blend(half, ob):
            def blend_body(g2, c2):
                g = half * 32 + g2
                fl = pl.ds(g * 16, 16)
                wx = wxb[fl]
                wy = wyb[fl]
                pv = g2 * 16 + i16
                for c in range(_C):
                    cc = jnp.full((16,), c, jnp.int32)
                    v00 = plsc.load_gather(rows[0][half], [pv, cc])
                    v01 = plsc.load_gather(rows[1][half], [pv, cc])
                    v10 = plsc.load_gather(rows[2][half], [pv, cc])
                    v11 = plsc.load_gather(rows[3][half], [pv, cc])
                    top = v00 + wx * (v01 - v00)
                    bot = v10 + wx * (v11 - v10)
                    res = top + wy * (bot - top)
                    ob[pl.ds(c * _SP + g * 16, 16)] = res
                return c2

            lax.fori_loop(0, 32, blend_body, 0)

        def omove(s, ob, sem, fire):
            b, yt, xt = coords(s)
            for c in range(_C):
                d = (pltpu.async_copy if fire else pltpu.make_async_copy)(
                    ob.at[pl.ds(c * _SP, _SP)], out_hbm.at[b, c, yt, xt], sem)
                if not fire:
                    d.wait()

        def do_strip(s, xb, ob, osem, it):
            idx_compute(xb)
            if _PROBE_GATHER:
                gathers(0, gs0, True)
                gathers(1, gs1, True)
            @pl.when(it > 0)
            def _():
                omove(s, ob, osem, False)

            if _PROBE_GATHER:
                gathers(0, gs0, False)
            if _PROBE_BLEND:
                blend(0, ob)
            if _PROBE_GATHER:
                gathers(1, gs1, False)
            if _PROBE_BLEND:
                blend(1, ob)
            omove(s, ob, osem, True)

        def pair(it, carry):
            s_a = wid * _SPW + 2 * it
            xmove(s_a + 1, xbb, xsb, True)
            xmove(s_a, xba, xsa, False)
            do_strip(s_a, xba, oba, osa, it)
            @pl.when(it < _SPW // 2 - 1)
            def _():
                xmove(s_a + 2, xba, xsa, True)

            xmove(s_a + 1, xbb, xsb, False)
            do_strip(s_a + 1, xbb, obb, osb, it)
            return carry

        xmove(wid * _SPW, xba, xsa, True)
        lax.fori_loop(0, _SPW // 2, pair, 0)
        omove(0, oba, osa, False)
        omove(0, obb, osb, False)

    return k(xf, table)


def kernel(x, texture):
    xf = x.reshape(4, 512, 4, 128, 2).transpose(0, 1, 2, 4, 3)
    tex_tiles = (texture.reshape(_C, 128, 8, 8, 128)
                 .transpose(0, 1, 3, 2, 4)
                 .reshape(_C, 128, 8, 1024))
    table = _clip_transpose(tex_tiles).reshape(_TEX * _TEX, _C)
    out5 = _sc_sample(xf, table)
    return (out5.reshape(4, _C, 64, 4, 8, 128)
            .transpose(0, 1, 2, 4, 3, 5)
            .reshape(4, _C, 512, 512))


# speedup vs baseline: 1.4602x; 1.4602x over previous
import functools

import jax
import jax.numpy as jnp
from jax import lax
from jax.experimental import pallas as pl
from jax.experimental.pallas import tpu as pltpu
from jax.experimental.pallas import tpu_sc as plsc

_C = 16
_TEX = 1024
_NPIX = 4 * 512 * 512
_NW = 32
_IPW = 512 * 512
_SP = 1024
_SPW = (_NPIX // _SP) // _NW
_CLIP_LO = -123.68
_CLIP_HI = 151.061

_SC_PARAMS = pltpu.CompilerParams(
    needs_layout_passes=False, use_tc_tiling_on_sc=False)

def _clip_transpose(tex_tiles):
    mesh = plsc.VectorSubcoreMesh(core_axis_name="c", subcore_axis_name="s")

    @functools.partial(
        pl.kernel,
        mesh=mesh,
        out_type=jax.ShapeDtypeStruct((_TEX, _TEX * _C), jnp.float32),
        compiler_params=_SC_PARAMS,
        scratch_types=[
            pltpu.VMEM((_C, 1024), jnp.float32),
            pltpu.VMEM((_C, 1024), jnp.float32),
            pltpu.VMEM((1024 * _C,), jnp.float32),
            pltpu.VMEM((1024 * _C,), jnp.float32),
            pltpu.SemaphoreType.DMA,
            pltpu.SemaphoreType.DMA,
            pltpu.SemaphoreType.DMA,
            pltpu.SemaphoreType.DMA,
        ],
    )
    def k(tex_hbm, tab_hbm, tba, tbb, oba, obb, isa, isb, osa, osb):
        wid = lax.axis_index("s") * 2 + lax.axis_index("c")
        i16 = lax.iota(jnp.int32, 16)

        def tin(blk, tb, sem, fire):
            yt = blk // 8
            xt = blk % 8
            for c in range(_C):
                d = (pltpu.async_copy if fire else pltpu.make_async_copy)(
                    tex_hbm.at[c, yt, xt], tb.at[c], sem)
                if not fire:
                    d.wait()

        def tout(blk, ob, sem, fire):
            yt = blk // 8
            xt = blk % 8
            for yin in range(8):
                d = (pltpu.async_copy if fire else pltpu.make_async_copy)(
                    ob.at[pl.ds(yin * 2048, 2048)],
                    tab_hbm.at[yt * 8 + yin, pl.ds(xt * 2048, 2048)], sem)
                if not fire:
                    d.wait()

        def transpose_block(tb, ob):
            def tr(g, c2):
                t16 = (g * 16 + i16) * _C
                for c in range(_C):
                    v = jnp.clip(tb[c, pl.ds(g * 16, 16)], _CLIP_LO, _CLIP_HI)
                    plsc.store_scatter(ob, [t16 + c], v)
                return c2

            lax.fori_loop(0, 64, tr, 0)

        def half_iter(blk, tb, ob, isem, osem, it):
            @pl.when(it > 0)
            def _():
                tout(blk, ob, osem, False)

            transpose_block(tb, ob)
            tout(blk, ob, osem, True)
            @pl.when(it < 15)
            def _():
                tin(blk + 2, tb, isem, True)

        def pair(it, carry):
            blk_a = wid * 32 + 2 * it
            tin(blk_a, tba, isa, False)
            half_iter(blk_a, tba, oba, isa, osa, it)
            tin(blk_a + 1, tbb, isb, False)
            half_iter(blk_a + 1, tbb, obb, isb, osb, it)
            return carry

        tin(wid * 32, tba, isa, True)
        tin(wid * 32 + 1, tbb, isb, True)
        lax.fori_loop(0, 16, pair, 0)
        tout(0, oba, osa, False)
        tout(0, obb, osb, False)

    return k(tex_tiles)


def _sc_sample(xf, table):
    mesh = plsc.VectorSubcoreMesh(core_axis_name="c", subcore_axis_name="s")
    _Q = _SP // 4

    @functools.partial(
        pl.kernel,
        mesh=mesh,
        out_type=jax.ShapeDtypeStruct((4, _C, 64, 4, _SP), jnp.float32),
        compiler_params=_SC_PARAMS,
        scratch_types=(
            [pltpu.VMEM((8, 2, 128), jnp.float32)] * 2
            + [pltpu.VMEM((8, 128), jnp.int32)] * 8
            + [pltpu.VMEM((_SP,), jnp.float32)] * 4
            + [pltpu.VMEM((_Q, _C), jnp.float32)] * 16
            + [pltpu.VMEM((_C * _SP,), jnp.float32)] * 2
            + [pltpu.SemaphoreType.DMA] * 8
        ),
    )
    def k(x_hbm, tab_hbm, out_hbm,
          xba, xbb,
          ia0, ia1, ia2, ia3, ib0, ib1, ib2, ib3,
          wxa, wya, wxbb, wybb,
          q00, q01, q02, q03, q10, q11, q12, q13,
          q20, q21, q22, q23, q30, q31, q32, q33,
          oba, obb,
          xsa, xsb, qs0, qs1, qs2, qs3, osa, osb):
        wid = lax.axis_index("s") * 2 + lax.axis_index("c")
        i16 = lax.iota(jnp.int32, 16)
        ch_base = i16 * _SP
        idxsets = ((ia0, ia1, ia2, ia3), (ib0, ib1, ib2, ib3))
        wsets = ((wxa, wya), (wxbb, wybb))
        qrows = ((q00, q01, q02, q03), (q10, q11, q12, q13),
                 (q20, q21, q22, q23), (q30, q31, q32, q33))
        qsems = (qs0, qs1, qs2, qs3)

        def coords(s):
            b = s // 256
            rem = s % 256
            return b, rem // 4, rem % 4

        def xmove(s, xb, sem, fire):
            b, yt, xt = coords(s)
            for yin in range(8):
                d = (pltpu.async_copy if fire else pltpu.make_async_copy)(
                    x_hbm.at[b, yt * 8 + yin, xt], xb.at[yin], sem)
                if not fire:
                    d.wait()

        def idx_compute(xb, iset, wset):
            i0, i1, i2, i3 = iset
            wxr, wyr = wset

            def idx_body(g, c2):
                yin = g // 8
                ks = pl.ds((g % 8) * 16, 16)
                gx = xb[yin, 0, ks]
                gy = xb[yin, 1, ks]
                ix = jnp.clip((gx + 1.0) * 0.5 * (_TEX - 1), 0.0, _TEX - 1.0)
                iy = jnp.clip((gy + 1.0) * 0.5 * (_TEX - 1), 0.0, _TEX - 1.0)
                x0 = ix.astype(jnp.int32)
                y0 = iy.astype(jnp.int32)
                wx = ix - x0.astype(jnp.float32)
                wy = iy - y0.astype(jnp.float32)
                x1 = jnp.minimum(x0 + 1, _TEX - 1)
                y1 = jnp.minimum(y0 + 1, _TEX - 1)
                row0 = y0 * _TEX
                row1 = y1 * _TEX
                j = g // 8
                sl = pl.ds((g % 8) * 16, 16)
                i0[j, sl] = row0 + x0
                i1[j, sl] = row0 + x1
                i2[j, sl] = row1 + x0
                i3[j, sl] = row1 + x1
                fl = pl.ds(g * 16, 16)
                wxr[fl] = wx
                wyr[fl] = wy
                return c2

            lax.fori_loop(0, 64, idx_body, 0)

        def qgather(iset, kq, fire):
            sem = qsems[kq]
            for jj in range(2):
                j = kq * 2 + jj
                rs = pl.ds(jj * 128, 128)
                for nb in range(4):
                    d = (pltpu.async_copy if fire else pltpu.make_async_copy)(
                        tab_hbm.at[iset[nb].at[j]], qrows[kq][nb].at[rs], sem)
                    if not fire:
                        d.wait()

        def blend_q(kq, wset, ob):
            wxr, wyr = wset

            def blend_body(g2, c2):
                g = kq * 16 + g2
                for i in range(16):
                    p = g * 16 + i
                    ph = g2 * 16 + i
                    zp = jnp.zeros((16,), jnp.int32) + p
                    wx = plsc.load_gather(wxr, [zp])
                    wy = plsc.load_gather(wyr, [zp])
                    v00 = qrows[kq][0][ph, :]
                    v01 = qrows[kq][1][ph, :]
                    v10 = qrows[kq][2][ph, :]
                    v11 = qrows[kq][3][ph, :]
                    top = v00 + wx * (v01 - v00)
                    bot = v10 + wx * (v11 - v10)
                    res = top + wy * (bot - top)
                    plsc.store_scatter(ob, [ch_base + p], res)
                return c2

            lax.fori_loop(0, 16, blend_body, 0)

        def omove(s, ob, sem, fire):
            b, yt, xt = coords(s)
            for c in range(_C):
                d = (pltpu.async_copy if fire else pltpu.make_async_copy)(
                    ob.at[pl.ds(c * _SP, _SP)], out_hbm.at[b, c, yt, xt], sem)
                if not fire:
                    d.wait()

        def do_strip(s, par, ob, osem, it, has_next):
            nxt = 1 - par

            @pl.when(has_next)
            def _():
                xmove(s + 1, (xba, xbb)[nxt], (xsa, xsb)[nxt], False)
                idx_compute((xba, xbb)[nxt], idxsets[nxt], wsets[nxt])

            for kq in range(4):
                qgather(idxsets[par], kq, False)
                blend_q(kq, wsets[par], ob)
                @pl.when(has_next)
                def _():
                    qgather(idxsets[nxt], kq, True)

            @pl.when(it > 0)
            def _():
                omove(s, ob, osem, False)

            omove(s, ob, osem, True)

        def pair(it, carry):
            s_a = wid * _SPW + 2 * it
            @pl.when(it < _SPW // 2 - 1)
            def _():
                xmove(s_a + 2, xba, xsa, True)

            do_strip(s_a, 0, oba, osa, it, it < _SPW // 2)
            @pl.when(it < _SPW // 2 - 1)
            def _():
                xmove(s_a + 3, xbb, xsb, True)

            do_strip(s_a + 1, 1, obb, osb, it, it < _SPW // 2 - 1)
            return carry

        s0 = wid * _SPW
        xmove(s0, xba, xsa, True)
        xmove(s0 + 1, xbb, xsb, True)
        xmove(s0, xba, xsa, False)
        idx_compute(xba, idxsets[0], wsets[0])
        for kq in range(4):
            qgather(idxsets[0], kq, True)
        lax.fori_loop(0, _SPW // 2, pair, 0)
        omove(0, oba, osa, False)
        omove(0, obb, osb, False)

    return k(xf, table)


def kernel(x, texture):
    xf = x.reshape(4, 512, 4, 128, 2).transpose(0, 1, 2, 4, 3)
    tex_tiles = (texture.reshape(_C, 128, 8, 8, 128)
                 .transpose(0, 1, 3, 2, 4)
                 .reshape(_C, 128, 8, 1024))
    table = _clip_transpose(tex_tiles).reshape(_TEX * _TEX, _C)
    out5 = _sc_sample(xf, table)
    return (out5.reshape(4, _C, 64, 4, 8, 128)
            .transpose(0, 1, 2, 4, 3, 5)
            .reshape(4, _C, 512, 512))
